# Initial kernel scaffold; baseline (speedup 1.0000x reference)
import jax
import jax.numpy as jnp
from jax import lax
from jax.experimental import pallas as pl
from jax.experimental.pallas import tpu as pltpu
from jax.experimental.pallas import tpu_sc as plsc

N = 10000
E = 320000
D_IN = 128
D_HID = 128
D_OUT = 64
DP = 144

NC = 2
NS = 16
NW = NC * NS
EPW = E // NW
CHUNK = 80
NCHUNK = EPW // CHUNK
ROWS_PER_TILE = N // NS
ZROWS = 125


def _agg_body(h_hbm, src_hbm, dst_hbm, z_hbm, out_hbm,
              sidx_v, didx_v, rows0, rows1, zbuf, acc_sh, gsem0, gsem1):
    cid = lax.axis_index("c")
    sid = lax.axis_index("s")
    wid = sid * NC + cid

    pltpu.sync_copy(z_hbm, zbuf)
    for j in range(ROWS_PER_TILE // ZROWS):
        pltpu.sync_copy(zbuf, acc_sh.at[pl.ds(sid * ROWS_PER_TILE + j * ZROWS, ZROWS)])

    pltpu.sync_copy(src_hbm.at[wid], sidx_v)
    pltpu.sync_copy(dst_hbm.at[wid], didx_v)
    plsc.subcore_barrier()

    def start_gather(c, buf, sem):
        pltpu.async_copy(h_hbm.at[sidx_v.at[c]], buf, sem)

    def wait_gather(c, buf, sem):
        pltpu.make_async_copy(h_hbm.at[sidx_v.at[c]], buf, sem).wait()

    def scatter(c, buf):
        pltpu.sync_copy(buf, acc_sh.at[didx_v.at[c]], add=True)

    start_gather(0, rows0, gsem0)

    def body(i, carry):
        c = 2 * i
        start_gather(c + 1, rows1, gsem1)
        wait_gather(c, rows0, gsem0)
        scatter(c, rows0)

        @pl.when(c + 2 < NCHUNK)
        def _():
            start_gather(c + 2, rows0, gsem0)

        wait_gather(c + 1, rows1, gsem1)
        scatter(c + 1, rows1)
        return carry

    lax.fori_loop(0, NCHUNK // 2, body, 0)
    wait_gather(NCHUNK - 1, rows0, gsem0)
    scatter(NCHUNK - 1, rows0)

    plsc.subcore_barrier()
    pltpu.sync_copy(acc_sh.at[pl.ds(sid * ROWS_PER_TILE, ROWS_PER_TILE)],
                    out_hbm.at[cid, pl.ds(sid * ROWS_PER_TILE, ROWS_PER_TILE)])


def _make_agg():
    mesh = plsc.VectorSubcoreMesh(core_axis_name="c", subcore_axis_name="s")
    return pl.kernel(
        _agg_body,
        out_type=jax.ShapeDtypeStruct((NC, N, DP), jnp.float32),
        mesh=mesh,
        scratch_types=[
            pltpu.VMEM((NCHUNK, CHUNK), jnp.int32),
            pltpu.VMEM((NCHUNK, CHUNK), jnp.int32),
            pltpu.VMEM((CHUNK, DP), jnp.float32),
            pltpu.VMEM((CHUNK, DP), jnp.float32),
            pltpu.VMEM((ZROWS, DP), jnp.float32),
            pltpu.VMEM_SHARED((N, DP), jnp.float32),
            pltpu.SemaphoreType.DMA,
            pltpu.SemaphoreType.DMA,
        ],
    )


def _layer1_body(acc_ref, feats_ref, ws_ref, wn_ref, b_ref, out_ref):
    s = acc_ref[0] + acc_ref[1]
    deg = jnp.maximum(s[:, D_IN], 1.0)
    hn = s[:, :D_IN] / deg[:, None]
    h = (jnp.dot(feats_ref[...], ws_ref[...], preferred_element_type=jnp.float32)
         + jnp.dot(hn, wn_ref[...], preferred_element_type=jnp.float32)
         + b_ref[...])
    h = jnp.maximum(h, 0.0)
    col = lax.broadcasted_iota(jnp.int32, (h.shape[0], DP - D_IN), 1)
    pad = jnp.where(col == 0, 1.0, 0.0).astype(jnp.float32)
    out_ref[...] = jnp.concatenate([h, pad], axis=1)


def _layer2_body(acc_ref, h_ref, ws_ref, wn_ref, b_ref, out_ref):
    s = acc_ref[0] + acc_ref[1]
    deg = jnp.maximum(s[:, D_IN], 1.0)
    hn = s[:, :D_IN] / deg[:, None]
    out_ref[...] = (jnp.dot(h_ref[:, :D_IN], ws_ref[...],
                            preferred_element_type=jnp.float32)
                    + jnp.dot(hn, wn_ref[...],
                              preferred_element_type=jnp.float32)
                    + b_ref[...])


BLK = 1000


def _make_layer1():
    grid = (N // BLK,)
    return pl.pallas_call(
        _layer1_body,
        grid=grid,
        in_specs=[
            pl.BlockSpec((NC, BLK, DP), lambda i: (0, i, 0)),
            pl.BlockSpec((BLK, D_IN), lambda i: (i, 0)),
            pl.BlockSpec((D_IN, D_HID), lambda i: (0, 0)),
            pl.BlockSpec((D_IN, D_HID), lambda i: (0, 0)),
            pl.BlockSpec((1, D_HID), lambda i: (0, 0)),
        ],
        out_specs=pl.BlockSpec((BLK, DP), lambda i: (i, 0)),
        out_shape=jax.ShapeDtypeStruct((N, DP), jnp.float32),
    )


def _make_layer2():
    grid = (N // BLK,)
    return pl.pallas_call(
        _layer2_body,
        grid=grid,
        in_specs=[
            pl.BlockSpec((NC, BLK, DP), lambda i: (0, i, 0)),
            pl.BlockSpec((BLK, DP), lambda i: (i, 0)),
            pl.BlockSpec((D_HID, D_OUT), lambda i: (0, 0)),
            pl.BlockSpec((D_HID, D_OUT), lambda i: (0, 0)),
            pl.BlockSpec((1, D_OUT), lambda i: (0, 0)),
        ],
        out_specs=pl.BlockSpec((BLK, D_OUT), lambda i: (i, 0)),
        out_shape=jax.ShapeDtypeStruct((N, D_OUT), jnp.float32),
    )


@jax.jit
def kernel(feats, edge_index, Ws1, Wn1, b1, Ws2, Wn2, b2):
    src = edge_index[0].astype(jnp.int32).reshape(NW, NCHUNK, CHUNK)
    dst = edge_index[1].astype(jnp.int32).reshape(NW, NCHUNK, CHUNK)
    pad = jnp.concatenate(
        [jnp.ones((N, 1), jnp.float32), jnp.zeros((N, DP - D_IN - 1), jnp.float32)],
        axis=1)
    feats_p = jnp.concatenate([feats, pad], axis=1)
    zeros = jnp.zeros((ZROWS, DP), jnp.float32)

    agg = _make_agg()
    acc1 = agg(feats_p, src, dst, zeros)
    h1p = _make_layer1()(acc1, feats, Ws1, Wn1, b1.reshape(1, D_HID))
    acc2 = agg(h1p, src, dst, zeros)
    out = _make_layer2()(acc2, h1p, Ws2, Wn2, b2.reshape(1, D_OUT))
    return out


# speedup vs baseline: 9.7555x; 9.7555x over previous
import jax
import jax.numpy as jnp
from jax import lax
from jax.experimental import pallas as pl
from jax.experimental.pallas import tpu as pltpu
from jax.experimental.pallas import tpu_sc as plsc

N = 10000
E = 320000
D_IN = 128
D_HID = 128
D_OUT = 64
DP = 144

NC = 2
NS = 16
NW = NC * NS
EPW = E // NW
CHUNK = 80
NCHUNK = EPW // CHUNK
ROWS_PER_TILE = N // NS


def _agg_body(h_hbm, src_hbm, dst_hbm, z_hbm, out_hbm,
              sidx_v, didx0, didx1, rows0, rows1, acc_sh,
              gsem0, gsem1, dsem0, dsem1):
    cid = lax.axis_index("c")
    sid = lax.axis_index("s")
    wid = sid * NC + cid

    pltpu.sync_copy(z_hbm, acc_sh.at[pl.ds(sid * ROWS_PER_TILE, ROWS_PER_TILE)])

    pltpu.sync_copy(src_hbm.at[wid], sidx_v)
    plsc.subcore_barrier()

    def start_chunk(c, buf, dbuf, gsem, dsem):
        pltpu.async_copy(h_hbm.at[sidx_v.at[c]], buf, gsem)
        pltpu.async_copy(dst_hbm.at[wid, c], dbuf, dsem)

    def wait_chunk(c, buf, dbuf, gsem, dsem):
        pltpu.make_async_copy(h_hbm.at[sidx_v.at[c]], buf, gsem).wait()
        pltpu.make_async_copy(dst_hbm.at[wid, c], dbuf, dsem).wait()

    def scatter(buf, dbuf):
        pltpu.sync_copy(buf, acc_sh.at[dbuf], add=True)

    start_chunk(0, rows0, didx0, gsem0, dsem0)

    def body(i, carry):
        c = 2 * i
        start_chunk(c + 1, rows1, didx1, gsem1, dsem1)
        wait_chunk(c, rows0, didx0, gsem0, dsem0)
        scatter(rows0, didx0)

        @pl.when(c + 2 < NCHUNK)
        def _():
            start_chunk(c + 2, rows0, didx0, gsem0, dsem0)

        wait_chunk(c + 1, rows1, didx1, gsem1, dsem1)
        scatter(rows1, didx1)
        return carry

    lax.fori_loop(0, NCHUNK // 2, body, 0)
    wait_chunk(NCHUNK - 1, rows0, didx0, gsem0, dsem0)
    scatter(rows0, didx0)

    plsc.subcore_barrier()
    pltpu.sync_copy(acc_sh.at[pl.ds(sid * ROWS_PER_TILE, ROWS_PER_TILE)],
                    out_hbm.at[cid, pl.ds(sid * ROWS_PER_TILE, ROWS_PER_TILE)])


def _make_agg():
    mesh = plsc.VectorSubcoreMesh(core_axis_name="c", subcore_axis_name="s")
    return pl.kernel(
        _agg_body,
        out_type=jax.ShapeDtypeStruct((NC, N, DP), jnp.float32),
        mesh=mesh,
        scratch_types=[
            pltpu.VMEM((NCHUNK, CHUNK), jnp.int32),
            pltpu.VMEM((CHUNK,), jnp.int32),
            pltpu.VMEM((CHUNK,), jnp.int32),
            pltpu.VMEM((CHUNK, DP), jnp.float32),
            pltpu.VMEM((CHUNK, DP), jnp.float32),
            pltpu.VMEM_SHARED((N, DP), jnp.float32),
            pltpu.SemaphoreType.DMA,
            pltpu.SemaphoreType.DMA,
            pltpu.SemaphoreType.DMA,
            pltpu.SemaphoreType.DMA,
        ],
        compiler_params=pltpu.CompilerParams(use_tc_tiling_on_sc=False),
    )


def _layer1_body(acc_ref, feats_ref, ws_ref, wn_ref, b_ref, out_ref):
    s = acc_ref[0] + acc_ref[1]
    deg = jnp.maximum(s[:, D_IN], 1.0)
    hn = s[:, :D_IN] / deg[:, None]
    h = (jnp.dot(feats_ref[...], ws_ref[...], preferred_element_type=jnp.float32)
         + jnp.dot(hn, wn_ref[...], preferred_element_type=jnp.float32)
         + b_ref[...])
    h = jnp.maximum(h, 0.0)
    col = lax.broadcasted_iota(jnp.int32, (h.shape[0], DP - D_IN), 1)
    pad = jnp.where(col == 0, 1.0, 0.0).astype(jnp.float32)
    out_ref[...] = jnp.concatenate([h, pad], axis=1)


def _layer2_body(acc_ref, h_ref, ws_ref, wn_ref, b_ref, out_ref):
    s = acc_ref[0] + acc_ref[1]
    deg = jnp.maximum(s[:, D_IN], 1.0)
    hn = s[:, :D_IN] / deg[:, None]
    out_ref[...] = (jnp.dot(h_ref[:, :D_IN], ws_ref[...],
                            preferred_element_type=jnp.float32)
                    + jnp.dot(hn, wn_ref[...],
                              preferred_element_type=jnp.float32)
                    + b_ref[...])


BLK = 1000


def _make_layer1():
    grid = (N // BLK,)
    return pl.pallas_call(
        _layer1_body,
        grid=grid,
        in_specs=[
            pl.BlockSpec((NC, BLK, DP), lambda i: (0, i, 0)),
            pl.BlockSpec((BLK, D_IN), lambda i: (i, 0)),
            pl.BlockSpec((D_IN, D_HID), lambda i: (0, 0)),
            pl.BlockSpec((D_IN, D_HID), lambda i: (0, 0)),
            pl.BlockSpec((1, D_HID), lambda i: (0, 0)),
        ],
        out_specs=pl.BlockSpec((BLK, DP), lambda i: (i, 0)),
        out_shape=jax.ShapeDtypeStruct((N, DP), jnp.float32),
    )


def _make_layer2():
    grid = (N // BLK,)
    return pl.pallas_call(
        _layer2_body,
        grid=grid,
        in_specs=[
            pl.BlockSpec((NC, BLK, DP), lambda i: (0, i, 0)),
            pl.BlockSpec((BLK, DP), lambda i: (i, 0)),
            pl.BlockSpec((D_HID, D_OUT), lambda i: (0, 0)),
            pl.BlockSpec((D_HID, D_OUT), lambda i: (0, 0)),
            pl.BlockSpec((1, D_OUT), lambda i: (0, 0)),
        ],
        out_specs=pl.BlockSpec((BLK, D_OUT), lambda i: (i, 0)),
        out_shape=jax.ShapeDtypeStruct((N, D_OUT), jnp.float32),
    )


@jax.jit
def kernel(feats, edge_index, Ws1, Wn1, b1, Ws2, Wn2, b2):
    src = edge_index[0].astype(jnp.int32).reshape(NW, NCHUNK, CHUNK)
    dst = edge_index[1].astype(jnp.int32).reshape(NW, NCHUNK, CHUNK)
    pad = jnp.concatenate(
        [jnp.ones((N, 1), jnp.float32), jnp.zeros((N, DP - D_IN - 1), jnp.float32)],
        axis=1)
    feats_p = jnp.concatenate([feats, pad], axis=1)
    zeros = jnp.zeros((ROWS_PER_TILE, DP), jnp.float32)

    agg = _make_agg()
    acc1 = agg(feats_p, src, dst, zeros)
    h1p = _make_layer1()(acc1, feats, Ws1, Wn1, b1.reshape(1, D_HID))
    acc2 = agg(h1p, src, dst, zeros)
    out = _make_layer2()(acc2, h1p, Ws2, Wn2, b2.reshape(1, D_OUT))
    return out
